# speedup vs baseline: 8.3927x; 8.3927x over previous
import functools

import jax
import jax.numpy as jnp
from jax import lax
from jax.experimental import pallas as pl
from jax.experimental.pallas import tpu as pltpu
from jax.experimental.pallas import tpu_sc as plsc

D = 128
IW = 128
G = 4


@functools.lru_cache(maxsize=None)
def _build(V, B):
    info = plsc.get_sparse_core_info()
    NW = info.num_cores * info.num_subcores
    rows_per_w = B // NW
    CH = G * IW
    n_chunks = rows_per_w // CH
    irows_per_w = rows_per_w // IW
    assert B % (NW * CH) == 0

    mesh = plsc.VectorSubcoreMesh(core_axis_name="c", subcore_axis_name="s")

    @functools.partial(
        pl.kernel,
        mesh=mesh,
        out_type=jax.ShapeDtypeStruct((B, D), jnp.float32),
        scratch_types=[
            pltpu.VMEM((G, IW), jnp.int32),
            pltpu.VMEM((CH, D), jnp.float32),
            pltpu.SemaphoreType.DMA,
        ],
    )
    def k(emb_hbm, idx_hbm, out_hbm, idx_v, rows_v, sem):
        wid = lax.axis_index("s") * info.num_cores + lax.axis_index("c")
        irow0 = wid * irows_per_w

        def chunk(i, carry):
            r = irow0 + i * G
            pltpu.sync_copy(idx_hbm.at[pl.ds(r, G)], idx_v)
            cps = [
                pltpu.async_copy(
                    emb_hbm.at[idx_v.at[j]],
                    rows_v.at[pl.ds(j * IW, IW)],
                    sem,
                )
                for j in range(G)
            ]
            for cp in cps:
                cp.wait()
            pltpu.sync_copy(rows_v, out_hbm.at[pl.ds(r * IW, CH)])
            return carry

        lax.fori_loop(0, n_chunks, chunk, 0)

    return k


def kernel(emb, token_id):
    flat = token_id.reshape(-1).astype(jnp.int32)
    B = flat.shape[0]
    idx2d = flat.reshape(B // IW, IW)
    return _build(emb.shape[0], B)(emb, idx2d)


# speedup vs baseline: 9.4739x; 1.1288x over previous
import functools

import jax
import jax.numpy as jnp
from jax import lax
from jax.experimental import pallas as pl
from jax.experimental.pallas import tpu as pltpu
from jax.experimental.pallas import tpu_sc as plsc

D = 128
IW = 128
NBUF = 4


@functools.lru_cache(maxsize=None)
def _build(V, B):
    info = plsc.get_sparse_core_info()
    NW = info.num_cores * info.num_subcores
    rows_per_w = B // NW
    n = rows_per_w // IW
    assert B % (NW * IW) == 0 and n % 4 == 0 and n >= 8

    mesh = plsc.VectorSubcoreMesh(core_axis_name="c", subcore_axis_name="s")

    @functools.partial(
        pl.kernel,
        mesh=mesh,
        out_type=jax.ShapeDtypeStruct((B, D), jnp.float32),
        scratch_types=(
            [pltpu.VMEM((n, IW), jnp.int32)]
            + [pltpu.VMEM((IW, D), jnp.float32) for _ in range(NBUF)]
            + [pltpu.SemaphoreType.DMA for _ in range(2 * NBUF)]
        ),
    )
    def k(emb_hbm, idx_hbm, out_hbm, idx_all, *bufs):
        rows_v = bufs[:NBUF]
        gsem = bufs[NBUF:2 * NBUF]
        osem = bufs[2 * NBUF:]
        wid = lax.axis_index("s") * info.num_cores + lax.axis_index("c")
        irow0 = wid * n

        def gather_cp(i, b):
            return pltpu.make_async_copy(
                emb_hbm.at[idx_all.at[i]], rows_v[b], gsem[b])

        def out_cp(i, b):
            return pltpu.make_async_copy(
                rows_v[b], out_hbm.at[pl.ds((irow0 + i) * IW, IW)], osem[b])

        pltpu.sync_copy(idx_hbm.at[pl.ds(irow0, n)], idx_all)

        gather_cp(0, 0).start()
        gather_cp(1, 1).start()
        for i in (0, 1):
            gather_cp(i, i).wait()
            out_cp(i, i).start()
            gather_cp(i + 2, i + 2).start()

        def quad(q, carry):
            i0 = 2 + q * 4
            for u in range(4):
                i = i0 + u
                b = (2 + u) % 4
                bf = u % 4
                gather_cp(i, b).wait()
                out_cp(i, b).start()
                out_cp(i - 2, bf).wait()
                gather_cp(i + 2, bf).start()
            return carry

        lax.fori_loop(0, (n - 4) // 4, quad, 0)

        for i in (n - 2, n - 1):
            b = i % 4
            gather_cp(i, b).wait()
            out_cp(i, b).start()
        for i in (n - 4, n - 3, n - 2, n - 1):
            out_cp(i, i % 4).wait()

    return k


def kernel(emb, token_id):
    flat = token_id.reshape(-1).astype(jnp.int32)
    B = flat.shape[0]
    idx2d = flat.reshape(B // IW, IW)
    return _build(emb.shape[0], B)(emb, idx2d)
